# Initial kernel scaffold; baseline (speedup 1.0000x reference)
import jax
import jax.numpy as jnp
from jax.experimental import pallas as pl


def kernel(indices, nids, data, weights):
    raise NotImplementedError("write your pallas kernel here")


# speedup vs baseline: 2.8925x; 2.8925x over previous
import functools

import jax
import jax.numpy as jnp
from jax import lax
from jax.experimental import pallas as pl
from jax.experimental.pallas import tpu as pltpu
from jax.experimental.pallas import tpu_sc as plsc

DATA_DIM = 28
DP = 32
N_NODES = 524288
N_CORNERS = 600000
N_QUERIES = 262144

NC = 2
NS = 16
L = 16
NW = NC * NS
QPW = N_QUERIES // NW
CHUNK = 128
NCHUNK = QPW // CHUNK


def _body(indices_hbm, nids_hbm, data_hbm, weights_hbm, out_hbm,
          idx_v, cid_v, cflat_v, rows_v, w_v, out_v, sem_n, sem_d):
    wid = lax.axis_index("s") * NC + lax.axis_index("c")
    base = wid * QPW

    iota = lax.iota(jnp.int32, L)
    hi = iota >> 3
    lo = iota & 7

    def chunk_body(g, _):
        qbase = pl.multiple_of(base + g * CHUNK, CHUNK)
        pltpu.sync_copy(indices_hbm.at[pl.ds(qbase, CHUNK)], idx_v)
        pltpu.async_copy(nids_hbm.at[idx_v], cid_v, sem_n).wait()
        for t in range(CHUNK // 2):
            idx_c = 2 * t + hi
            cvec = plsc.load_gather(cid_v, [idx_c, lo])
            cflat_v[t // 8, pl.ds((t % 8) * L, L)] = cvec
        copies = [
            pltpu.async_copy(data_hbm.at[cflat_v.at[k]], rows_v.at[k], sem_d)
            for k in range(8)
        ]
        for c in copies:
            c.wait()
        pltpu.sync_copy(
            weights_hbm.at[pl.ds(pl.multiple_of(qbase // 2, CHUNK // 2), CHUNK // 2), :],
            w_v)

        def q_body(c2, _):
            k = c2 // 8
            m = (c2 % 8) * L
            wv = w_v[c2, :]
            for h, c in ((0, 2 * c2), (8, 2 * c2 + 1)):
                w0 = wv[h]
                acc_lo = w0 * rows_v[k, m + h, pl.ds(0, L)]
                acc_hi = w0 * rows_v[k, m + h, pl.ds(L, L)]
                for j in range(1, 8):
                    wj = wv[h + j]
                    acc_lo = acc_lo + wj * rows_v[k, m + h + j, pl.ds(0, L)]
                    acc_hi = acc_hi + wj * rows_v[k, m + h + j, pl.ds(L, L)]
                out_v[c, pl.ds(0, L)] = acc_lo
                out_v[c, pl.ds(L, L)] = acc_hi
            return 0

        lax.fori_loop(0, CHUNK // 2, q_body, 0)
        pltpu.sync_copy(out_v, out_hbm.at[pl.ds(qbase, CHUNK), :])
        return 0

    lax.fori_loop(0, NCHUNK, chunk_body, 0)


@jax.jit
def kernel(indices, nids, data, weights):
    mesh = plsc.VectorSubcoreMesh(core_axis_name="c", subcore_axis_name="s")
    run = functools.partial(
        pl.kernel,
        mesh=mesh,
        out_type=jax.ShapeDtypeStruct((N_QUERIES, DP), jnp.float32),
        compiler_params=pltpu.CompilerParams(
            needs_layout_passes=False, use_tc_tiling_on_sc=False),
        scratch_types=[
            pltpu.VMEM((CHUNK,), jnp.int32),
            pltpu.VMEM((CHUNK, 8), jnp.int32),
            pltpu.VMEM((8, CHUNK), jnp.int32),
            pltpu.VMEM((8, CHUNK, DP), jnp.float32),
            pltpu.VMEM((CHUNK // 2, L), jnp.float32),
            pltpu.VMEM((CHUNK, DP), jnp.float32),
            pltpu.SemaphoreType.DMA,
            pltpu.SemaphoreType.DMA,
        ],
    )(_body)
    data_p = jnp.concatenate(
        [data, jnp.zeros((N_CORNERS, DP - DATA_DIM), jnp.float32)], axis=1)
    out = run(indices, nids, data_p, weights.reshape(N_QUERIES // 2, 2 * 8))
    return out[:, :DATA_DIM]
